# baseline (speedup 1.0000x reference)
import functools

import jax
import jax.numpy as jnp
from jax import lax
from jax.experimental import pallas as pl
from jax.experimental.pallas import tpu as pltpu
from jax.experimental.pallas import tpu_sc as plsc

L = 16
NC = 2
NS = 16
ACC = 8


def _sc_argmax_body(x_hbm, idx_hbm, xb0, xb1, xb2, ibuf, sem0, sem1, sem2, *, n_rows, n_cols):
    nw = NC * NS
    rows_per_w = n_rows // nw
    wid = lax.axis_index("s") * NC + lax.axis_index("c")
    base = wid * rows_per_w
    n_chunks = n_cols // L
    n_blocks = n_chunks // ACC
    iota = lax.iota(jnp.int32, L)
    xbufs = [xb0, xb1, xb2]
    sems = [sem0, sem1, sem2]

    cps = [
        pltpu.async_copy(x_hbm.at[base + r], xbufs[r], sems[r])
        for r in range(rows_per_w)
    ]
    gvec = jnp.zeros((L,), jnp.int32)
    for r in range(rows_per_w):
        cps[r].wait()
        xrow = xbufs[r]

        def body(i, carry):
            bests, bblks = carry
            ib = jnp.full((L,), 0, jnp.int32) + i
            new_bests = []
            new_bblks = []
            for k in range(ACC):
                v = xrow[pl.ds((i * ACC + k) * L, L)]
                gt = v > bests[k]
                new_bests.append(jnp.where(gt, v, bests[k]))
                new_bblks.append(jnp.where(gt, ib, bblks[k]))
            return tuple(new_bests), tuple(new_bblks)

        best0 = tuple(jnp.full((L,), -jnp.inf, jnp.float32) for _ in range(ACC))
        bblk0 = tuple(jnp.zeros((L,), jnp.int32) for _ in range(ACC))
        bests, bblks = lax.fori_loop(0, n_blocks, body, (best0, bblk0))

        pairs = [
            (bests[k], bblks[k] * (ACC * L) + (iota + k * L)) for k in range(ACC)
        ]
        while len(pairs) > 1:
            nxt_pairs = []
            for a in range(0, len(pairs), 2):
                v1, i1 = pairs[a]
                v2, i2 = pairs[a + 1]
                better = (v2 > v1) | ((v2 == v1) & (i2 < i1))
                nxt_pairs.append(
                    (jnp.where(better, v2, v1), jnp.where(better, i2, i1))
                )
            pairs = nxt_pairs
        best, bidx = pairs[0]

        sbits = lax.bitcast_convert_type(best, jnp.int32)
        skey = sbits ^ ((sbits >> 31) & jnp.int32(0x7FFFFFFF))
        m = jnp.int32(-(2**31))
        gidx = jnp.int32(2**31 - 1)
        for j in range(L):
            k = skey[j]
            b = bidx[j]
            better = (k > m) | ((k == m) & (b < gidx))
            m = jnp.where(better, k, m)
            gidx = jnp.where(better, b, gidx)

        gvec = jnp.where(iota == r, gidx, gvec)

    ibuf[...] = gvec
    pltpu.sync_copy(ibuf, idx_hbm.at[wid])


def _tc_zero_body(o_ref):
    o_ref[...] = jnp.zeros_like(o_ref)


def _tc_argmax_body(x_ref, o_ref, *, n_cols):
    x = x_ref[...]
    m = jnp.max(x, axis=1, keepdims=True)
    io = lax.broadcasted_iota(jnp.int32, x.shape, 1)
    masked = jnp.where(x == m, io, jnp.int32(2**31 - 1))
    o_ref[...] = jnp.min(masked, axis=1).reshape(1, 1, 8)


def _tc_scatter_body(idx_sc_ref, idx_tc_ref, z_hbm, o_hbm, ident, sem, *, n_rows, sc_rows, rows_per_w):
    r_io = lax.broadcasted_iota(jnp.int32, (8, 128), 0)
    c_io = lax.broadcasted_iota(jnp.int32, (8, 128), 1)
    for t in range(n_rows // 8):
        ident[pl.ds(t * 8, 8), :] = (r_io + t * 8 == c_io).astype(jnp.float32)
    for i in range(n_rows):
        if i < sc_rows:
            col = idx_sc_ref[i // rows_per_w, i % rows_per_w]
        else:
            j = i - sc_rows
            col = idx_tc_ref[j // 8, 0, j % 8]
        start = (col // 128) * 128
        lane = col - start
        pltpu.make_async_copy(
            ident.at[pl.ds(lane, 1), :],
            o_hbm.at[pl.ds(i, 1), pl.ds(start, 128)],
            sem,
        ).start()
    pltpu.make_async_copy(
        z_hbm.at[pl.ds(0, n_rows), pl.ds(0, 128)], ident, sem
    ).wait()


def kernel(input):
    n_rows, n_cols = input.shape
    nw = NC * NS
    tc_rows = 32
    sc_rows = n_rows - tc_rows
    rows_per_w = sc_rows // nw

    mesh = plsc.VectorSubcoreMesh(
        core_axis_name="c", subcore_axis_name="s", num_cores=NC, num_subcores=NS
    )
    idx2d = pl.kernel(
        functools.partial(_sc_argmax_body, n_rows=sc_rows, n_cols=n_cols),
        out_type=jax.ShapeDtypeStruct((nw, L), jnp.int32),
        mesh=mesh,
        scratch_types=[
            pltpu.VMEM((n_cols,), jnp.float32),
            pltpu.VMEM((n_cols,), jnp.float32),
            pltpu.VMEM((n_cols,), jnp.float32),
            pltpu.VMEM((L,), jnp.int32),
            pltpu.SemaphoreType.DMA,
            pltpu.SemaphoreType.DMA,
            pltpu.SemaphoreType.DMA,
        ],
    )(input)

    zeros = pl.pallas_call(
        _tc_zero_body,
        out_shape=jax.ShapeDtypeStruct((n_rows, n_cols), jnp.float32),
        grid=(4,),
        out_specs=pl.BlockSpec((n_rows // 4, n_cols), lambda i: (i, 0)),
    )()

    idx_tc = pl.pallas_call(
        functools.partial(_tc_argmax_body, n_cols=n_cols),
        grid=(tc_rows // 8,),
        in_specs=[pl.BlockSpec((8, n_cols), lambda i: (i + sc_rows // 8, 0))],
        out_specs=pl.BlockSpec((1, 1, 8), lambda i: (i, 0, 0)),
        out_shape=jax.ShapeDtypeStruct((tc_rows // 8, 1, 8), jnp.int32),
    )(input)

    out = pl.pallas_call(
        functools.partial(
            _tc_scatter_body,
            n_rows=n_rows,
            sc_rows=sc_rows,
            rows_per_w=rows_per_w,
        ),
        in_specs=[
            pl.BlockSpec(memory_space=pltpu.SMEM),
            pl.BlockSpec(memory_space=pltpu.SMEM),
            pl.BlockSpec(memory_space=pltpu.HBM),
        ],
        out_specs=pl.BlockSpec(memory_space=pltpu.HBM),
        out_shape=jax.ShapeDtypeStruct((n_rows, n_cols), jnp.float32),
        input_output_aliases={2: 0},
        scratch_shapes=[
            pltpu.VMEM((n_rows, 128), jnp.float32),
            pltpu.SemaphoreType.DMA,
        ],
    )(idx2d, idx_tc, zeros)
    return out


# speedup vs baseline: 1.0085x; 1.0085x over previous
import functools

import jax
import jax.numpy as jnp
from jax import lax
from jax.experimental import pallas as pl
from jax.experimental.pallas import tpu as pltpu
from jax.experimental.pallas import tpu_sc as plsc

L = 16
NC = 2
NS = 16
ACC = 8


def _sc_argmax_body(x_hbm, idx_hbm, xb0, xb1, ibuf, sem0, sem1, *, n_rows, n_cols):
    nw = NC * NS
    rows_per_w = n_rows // nw
    wid = lax.axis_index("s") * NC + lax.axis_index("c")
    base = wid * rows_per_w
    n_chunks = n_cols // L
    n_blocks = n_chunks // ACC
    iota = lax.iota(jnp.int32, L)
    xbufs = [xb0, xb1]
    sems = [sem0, sem1]

    cp = pltpu.async_copy(x_hbm.at[base], xb0, sem0)
    gvec = jnp.zeros((L,), jnp.int32)
    for r in range(rows_per_w):
        nxt = None
        if r + 1 < rows_per_w:
            nxt = pltpu.async_copy(
                x_hbm.at[base + r + 1], xbufs[(r + 1) % 2], sems[(r + 1) % 2]
            )
        cp.wait()
        xrow = xbufs[r % 2]

        def body(i, carry):
            bests, bblks = carry
            ib = jnp.full((L,), 0, jnp.int32) + i
            new_bests = []
            new_bblks = []
            for k in range(ACC):
                v = xrow[pl.ds((i * ACC + k) * L, L)]
                gt = v > bests[k]
                new_bests.append(jnp.where(gt, v, bests[k]))
                new_bblks.append(jnp.where(gt, ib, bblks[k]))
            return tuple(new_bests), tuple(new_bblks)

        best0 = tuple(jnp.full((L,), -jnp.inf, jnp.float32) for _ in range(ACC))
        bblk0 = tuple(jnp.zeros((L,), jnp.int32) for _ in range(ACC))
        bests, bblks = lax.fori_loop(0, n_blocks, body, (best0, bblk0))

        pairs = [
            (bests[k], bblks[k] * (ACC * L) + (iota + k * L)) for k in range(ACC)
        ]
        while len(pairs) > 1:
            nxt_pairs = []
            for a in range(0, len(pairs), 2):
                v1, i1 = pairs[a]
                v2, i2 = pairs[a + 1]
                better = (v2 > v1) | ((v2 == v1) & (i2 < i1))
                nxt_pairs.append(
                    (jnp.where(better, v2, v1), jnp.where(better, i2, i1))
                )
            pairs = nxt_pairs
        best, bidx = pairs[0]

        sbits = lax.bitcast_convert_type(best, jnp.int32)
        skey = sbits ^ ((sbits >> 31) & jnp.int32(0x7FFFFFFF))
        m = jnp.int32(-(2**31))
        gidx = jnp.int32(2**31 - 1)
        for j in range(L):
            k = skey[j]
            b = bidx[j]
            better = (k > m) | ((k == m) & (b < gidx))
            m = jnp.where(better, k, m)
            gidx = jnp.where(better, b, gidx)

        gvec = jnp.where(iota == r, gidx, gvec)
        cp = nxt

    ibuf[...] = gvec
    pltpu.sync_copy(ibuf, idx_hbm.at[wid])


def _tc_zero_body(o_ref):
    o_ref[...] = jnp.zeros_like(o_ref)


def _tc_argmax_body(x_ref, o_ref, *, n_cols):
    x = x_ref[...]
    m = jnp.max(x, axis=1, keepdims=True)
    io = lax.broadcasted_iota(jnp.int32, x.shape, 1)
    masked = jnp.where(x == m, io, jnp.int32(2**31 - 1))
    o_ref[...] = jnp.min(masked, axis=1).reshape(1, 1, 8)


def _tc_scatter_body(idx_sc_ref, idx_tc_ref, z_hbm, o_hbm, ident, sem, *, n_rows, sc_rows, rows_per_w):
    r_io = lax.broadcasted_iota(jnp.int32, (8, 128), 0)
    c_io = lax.broadcasted_iota(jnp.int32, (8, 128), 1)
    for t in range(n_rows // 8):
        ident[pl.ds(t * 8, 8), :] = (r_io + t * 8 == c_io).astype(jnp.float32)
    for i in range(n_rows):
        if i < sc_rows:
            col = idx_sc_ref[i // rows_per_w, i % rows_per_w]
        else:
            j = i - sc_rows
            col = idx_tc_ref[j // 8, 0, j % 8]
        start = (col // 128) * 128
        lane = col - start
        pltpu.make_async_copy(
            ident.at[pl.ds(lane, 1), :],
            o_hbm.at[pl.ds(i, 1), pl.ds(start, 128)],
            sem,
        ).start()
    pltpu.make_async_copy(
        z_hbm.at[pl.ds(0, n_rows), pl.ds(0, 128)], ident, sem
    ).wait()


def kernel(input):
    n_rows, n_cols = input.shape
    nw = NC * NS
    tc_rows = 32
    sc_rows = n_rows - tc_rows
    rows_per_w = sc_rows // nw

    mesh = plsc.VectorSubcoreMesh(
        core_axis_name="c", subcore_axis_name="s", num_cores=NC, num_subcores=NS
    )
    idx2d = pl.kernel(
        functools.partial(_sc_argmax_body, n_rows=sc_rows, n_cols=n_cols),
        out_type=jax.ShapeDtypeStruct((nw, L), jnp.int32),
        mesh=mesh,
        scratch_types=[
            pltpu.VMEM((n_cols,), jnp.float32),
            pltpu.VMEM((n_cols,), jnp.float32),
            pltpu.VMEM((L,), jnp.int32),
            pltpu.SemaphoreType.DMA,
            pltpu.SemaphoreType.DMA,
        ],
    )(input)

    zeros = pl.pallas_call(
        _tc_zero_body,
        out_shape=jax.ShapeDtypeStruct((n_rows, n_cols), jnp.float32),
        grid=(4,),
        out_specs=pl.BlockSpec((n_rows // 4, n_cols), lambda i: (i, 0)),
    )()

    idx_tc = pl.pallas_call(
        functools.partial(_tc_argmax_body, n_cols=n_cols),
        grid=(tc_rows // 8,),
        in_specs=[pl.BlockSpec((8, n_cols), lambda i: (i + sc_rows // 8, 0))],
        out_specs=pl.BlockSpec((1, 1, 8), lambda i: (i, 0, 0)),
        out_shape=jax.ShapeDtypeStruct((tc_rows // 8, 1, 8), jnp.int32),
    )(input)

    out = pl.pallas_call(
        functools.partial(
            _tc_scatter_body,
            n_rows=n_rows,
            sc_rows=sc_rows,
            rows_per_w=rows_per_w,
        ),
        in_specs=[
            pl.BlockSpec(memory_space=pltpu.SMEM),
            pl.BlockSpec(memory_space=pltpu.SMEM),
            pl.BlockSpec(memory_space=pltpu.HBM),
        ],
        out_specs=pl.BlockSpec(memory_space=pltpu.HBM),
        out_shape=jax.ShapeDtypeStruct((n_rows, n_cols), jnp.float32),
        input_output_aliases={2: 0},
        scratch_shapes=[
            pltpu.VMEM((n_rows, 128), jnp.float32),
            pltpu.SemaphoreType.DMA,
        ],
    )(idx2d, idx_tc, zeros)
    return out


# speedup vs baseline: 1.0108x; 1.0023x over previous
import functools

import jax
import jax.numpy as jnp
from jax import lax
from jax.experimental import pallas as pl
from jax.experimental.pallas import tpu as pltpu
from jax.experimental.pallas import tpu_sc as plsc

L = 16
NC = 2
NS = 16
ACC = 8


def _sc_argmax_body(x_hbm, idx_hbm, xb0, xb1, ibuf, sem0, sem1, *, n_rows, n_cols):
    nw = NC * NS
    rows_per_w = n_rows // nw
    wid = lax.axis_index("s") * NC + lax.axis_index("c")
    base = wid * rows_per_w
    n_chunks = n_cols // L
    n_blocks = n_chunks // ACC
    iota = lax.iota(jnp.int32, L)
    xbufs = [xb0, xb1]
    sems = [sem0, sem1]

    cp = pltpu.async_copy(x_hbm.at[base], xb0, sem0)
    gvec = jnp.zeros((L,), jnp.int32)
    for r in range(rows_per_w):
        nxt = None
        if r + 1 < rows_per_w:
            nxt = pltpu.async_copy(
                x_hbm.at[base + r + 1], xbufs[(r + 1) % 2], sems[(r + 1) % 2]
            )
        cp.wait()
        xrow = xbufs[r % 2]

        def body(i, carry):
            bests, bblks = carry
            ib = jnp.full((L,), 0, jnp.int32) + i
            new_bests = []
            new_bblks = []
            for k in range(ACC):
                v = xrow[pl.ds((i * ACC + k) * L, L)]
                gt = v > bests[k]
                new_bests.append(jnp.where(gt, v, bests[k]))
                new_bblks.append(jnp.where(gt, ib, bblks[k]))
            return tuple(new_bests), tuple(new_bblks)

        best0 = tuple(jnp.full((L,), -jnp.inf, jnp.float32) for _ in range(ACC))
        bblk0 = tuple(jnp.zeros((L,), jnp.int32) for _ in range(ACC))
        bests, bblks = lax.fori_loop(0, n_blocks, body, (best0, bblk0), unroll=2)

        pairs = [
            (bests[k], bblks[k] * (ACC * L) + (iota + k * L)) for k in range(ACC)
        ]
        while len(pairs) > 1:
            nxt_pairs = []
            for a in range(0, len(pairs), 2):
                v1, i1 = pairs[a]
                v2, i2 = pairs[a + 1]
                better = (v2 > v1) | ((v2 == v1) & (i2 < i1))
                nxt_pairs.append(
                    (jnp.where(better, v2, v1), jnp.where(better, i2, i1))
                )
            pairs = nxt_pairs
        best, bidx = pairs[0]

        sbits = lax.bitcast_convert_type(best, jnp.int32)
        skey = sbits ^ ((sbits >> 31) & jnp.int32(0x7FFFFFFF))
        m = jnp.int32(-(2**31))
        gidx = jnp.int32(2**31 - 1)
        for j in range(L):
            k = skey[j]
            b = bidx[j]
            better = (k > m) | ((k == m) & (b < gidx))
            m = jnp.where(better, k, m)
            gidx = jnp.where(better, b, gidx)

        gvec = jnp.where(iota == r, gidx, gvec)
        cp = nxt

    ibuf[...] = gvec
    pltpu.sync_copy(ibuf, idx_hbm.at[wid])


def _tc_zero_body(o_ref):
    o_ref[...] = jnp.zeros_like(o_ref)


def _tc_argmax_body(x_ref, o_ref, *, n_cols):
    x = x_ref[...]
    m = jnp.max(x, axis=1, keepdims=True)
    io = lax.broadcasted_iota(jnp.int32, x.shape, 1)
    masked = jnp.where(x == m, io, jnp.int32(2**31 - 1))
    o_ref[...] = jnp.min(masked, axis=1).reshape(1, 1, 8)


def _tc_scatter_body(idx_sc_ref, idx_tc_ref, z_hbm, o_hbm, ident, sem, *, n_rows, sc_rows, rows_per_w):
    r_io = lax.broadcasted_iota(jnp.int32, (8, 128), 0)
    c_io = lax.broadcasted_iota(jnp.int32, (8, 128), 1)
    for t in range(n_rows // 8):
        ident[pl.ds(t * 8, 8), :] = (r_io + t * 8 == c_io).astype(jnp.float32)
    for i in range(n_rows):
        if i < sc_rows:
            col = idx_sc_ref[i // rows_per_w, i % rows_per_w]
        else:
            j = i - sc_rows
            col = idx_tc_ref[j // 8, 0, j % 8]
        start = (col // 128) * 128
        lane = col - start
        pltpu.make_async_copy(
            ident.at[pl.ds(lane, 1), :],
            o_hbm.at[pl.ds(i, 1), pl.ds(start, 128)],
            sem,
        ).start()
    pltpu.make_async_copy(
        z_hbm.at[pl.ds(0, n_rows), pl.ds(0, 128)], ident, sem
    ).wait()


def kernel(input):
    n_rows, n_cols = input.shape
    nw = NC * NS
    tc_rows = 32
    sc_rows = n_rows - tc_rows
    rows_per_w = sc_rows // nw

    mesh = plsc.VectorSubcoreMesh(
        core_axis_name="c", subcore_axis_name="s", num_cores=NC, num_subcores=NS
    )
    idx2d = pl.kernel(
        functools.partial(_sc_argmax_body, n_rows=sc_rows, n_cols=n_cols),
        out_type=jax.ShapeDtypeStruct((nw, L), jnp.int32),
        mesh=mesh,
        scratch_types=[
            pltpu.VMEM((n_cols,), jnp.float32),
            pltpu.VMEM((n_cols,), jnp.float32),
            pltpu.VMEM((L,), jnp.int32),
            pltpu.SemaphoreType.DMA,
            pltpu.SemaphoreType.DMA,
        ],
    )(input)

    zeros = pl.pallas_call(
        _tc_zero_body,
        out_shape=jax.ShapeDtypeStruct((n_rows, n_cols), jnp.float32),
        grid=(4,),
        out_specs=pl.BlockSpec((n_rows // 4, n_cols), lambda i: (i, 0)),
    )()

    idx_tc = pl.pallas_call(
        functools.partial(_tc_argmax_body, n_cols=n_cols),
        grid=(tc_rows // 8,),
        in_specs=[pl.BlockSpec((8, n_cols), lambda i: (i + sc_rows // 8, 0))],
        out_specs=pl.BlockSpec((1, 1, 8), lambda i: (i, 0, 0)),
        out_shape=jax.ShapeDtypeStruct((tc_rows // 8, 1, 8), jnp.int32),
    )(input)

    out = pl.pallas_call(
        functools.partial(
            _tc_scatter_body,
            n_rows=n_rows,
            sc_rows=sc_rows,
            rows_per_w=rows_per_w,
        ),
        in_specs=[
            pl.BlockSpec(memory_space=pltpu.SMEM),
            pl.BlockSpec(memory_space=pltpu.SMEM),
            pl.BlockSpec(memory_space=pltpu.HBM),
        ],
        out_specs=pl.BlockSpec(memory_space=pltpu.HBM),
        out_shape=jax.ShapeDtypeStruct((n_rows, n_cols), jnp.float32),
        input_output_aliases={2: 0},
        scratch_shapes=[
            pltpu.VMEM((n_rows, 128), jnp.float32),
            pltpu.SemaphoreType.DMA,
        ],
    )(idx2d, idx_tc, zeros)
    return out
